# baseline (speedup 1.0000x reference)
import jax
import jax.numpy as jnp
from jax.experimental import pallas as pl

B, S, D = 32, 512, 256
H = 8
DH = D // H
DFF = 1024
NEG = -1e30
BF = jnp.bfloat16


def _dot(a, b):
    return jax.lax.dot_general(a, b, (((1,), (1,)), ((), ())),
                               preferred_element_type=jnp.float32)


def _ln(x, s, b):
    m = jnp.mean(x, axis=1, keepdims=True)
    v = jnp.mean((x - m) * (x - m), axis=1, keepdims=True)
    return (x - m) * jax.lax.rsqrt(v + 1e-5) * s + b


def _body(x_ref, am_ref, Ww_ref, bw_ref, Wa1_ref, ba1_ref, Wa2_ref, ba2_ref,
          ln1s_ref, ln1b_ref, Wqkv_ref, bqkv_ref, Wo_ref, bo_ref,
          ln2s_ref, ln2b_ref, Wm1_ref, bm1_ref, Wm2_ref, bm2_ref,
          out_ref, avg_ref):
    b_idx = pl.program_id(0)
    x = x_ref[0]
    ami = am_ref[0, 0]

    w = jnp.sum(x * Ww_ref[...], axis=1, keepdims=True) + bw_ref[0, 0]
    a1 = _dot(x, Wa1_ref[...]) + ba1_ref[...]
    a1 = a1 / (1.0 + jnp.exp(-a1))
    lcol = _dot(a1, Wa2_ref[...]) + ba2_ref[...]
    mask_col = lcol[:, 1:2] > lcol[:, 0:1]
    mask_f = mask_col.astype(jnp.float32)
    cnt = jnp.sum(mask_f, axis=0, keepdims=True)
    nmask = jnp.float32(S) - cnt

    a = _ln(x, ln1s_ref[...], ln1b_ref[...]).astype(BF)
    qkv = _dot(a, Wqkv_ref[...]) + bqkv_ref[...]

    scale = jnp.float32(1.0 / (DH ** 0.5))
    o_heads = []
    for h in range(H):
        q = (qkv[:, h * DH:(h + 1) * DH] * scale).astype(BF)
        k = (qkv[:, D + h * DH:D + (h + 1) * DH] * mask_f).astype(BF)
        v = (qkv[:, 2 * D + h * DH:2 * D + (h + 1) * DH] * mask_f).astype(BF)
        p = jnp.exp(_dot(q, k))
        r = 1.0 / (jnp.sum(p, axis=1, keepdims=True) - nmask)
        o_heads.append(jax.lax.dot_general(
            p.astype(BF), v, (((1,), (0,)), ((), ())),
            preferred_element_type=jnp.float32) * r)
    o = jnp.concatenate(o_heads, axis=1).astype(BF)

    h1 = x + _dot(o, Wo_ref[...]) + bo_ref[...]
    m = _ln(h1, ln2s_ref[...], ln2b_ref[...]).astype(BF)
    g = _dot(m, Wm1_ref[...]) + bm1_ref[...]
    g = 0.5 * g * (1.0 + jnp.tanh(0.7978845608028654 * (g + 0.044715 * g * g * g)))
    h2 = h1 + _dot(g.astype(BF), Wm2_ref[...]) + bm2_ref[...]

    out_ref[0] = jnp.where(mask_col, h2 * w, x)

    @pl.when(b_idx == 0)
    def _():
        avg_ref[...] = jnp.zeros((1, 1), jnp.float32)
    avg_ref[...] += cnt * jnp.float32(1.0 / B)


def kernel(x, attention_mask, Ww, bw, Wk1, bk1, Wk2, bk2, Wa1, ba1, Wa2, ba2,
           ln1_s, ln1_b, Wqkv, bqkv, Wo, bo, ln2_s, ln2_b, Wm1, bm1, Wm2, bm2):
    del Wk1, bk1, Wk2, bk2

    full = lambda shape: pl.BlockSpec(shape, lambda b: (0,) * len(shape))
    in_specs = [
        pl.BlockSpec((1, S, D), lambda b: (b, 0, 0)),
        pl.BlockSpec((1, 1, 1, S), lambda b: (b, 0, 0, 0)),
        full((1, D)),
        full((1, 1)),
        full((D // 2, D)),
        full((1, D // 2)),
        full((2, D // 2)),
        full((1, 2)),
        full((1, D)),
        full((1, D)),
        full((3 * D, D)),
        full((1, 3 * D)),
        full((D, D)),
        full((1, D)),
        full((1, D)),
        full((1, D)),
        full((DFF, D)),
        full((1, DFF)),
        full((D, DFF)),
        full((1, D)),
    ]
    out_specs = [
        pl.BlockSpec((1, S, D), lambda b: (b, 0, 0)),
        pl.BlockSpec((1, 1), lambda b: (0, 0)),
    ]
    out, avg = pl.pallas_call(
        _body,
        grid=(B,),
        in_specs=in_specs,
        out_specs=out_specs,
        out_shape=[
            jax.ShapeDtypeStruct((B, S, D), jnp.float32),
            jax.ShapeDtypeStruct((1, 1), jnp.float32),
        ],
    )(x, attention_mask,
      Ww, bw.reshape(1, 1), Wa1, ba1.reshape(1, -1), Wa2, ba2.reshape(1, -1),
      ln1_s.reshape(1, -1), ln1_b.reshape(1, -1),
      Wqkv.astype(BF), bqkv.reshape(1, -1),
      Wo.astype(BF), bo.reshape(1, -1),
      ln2_s.reshape(1, -1), ln2_b.reshape(1, -1),
      Wm1.astype(BF), bm1.reshape(1, -1),
      Wm2.astype(BF), bm2.reshape(1, -1))
    return (out, avg.reshape(()))


# speedup vs baseline: 1.0002x; 1.0002x over previous
import jax
import jax.numpy as jnp
from jax.experimental import pallas as pl

B, S, D = 32, 512, 256
H = 8
DH = D // H
DFF = 1024
NEG = -1e30
BF = jnp.bfloat16


def _dot(a, b):
    return jax.lax.dot_general(a, b, (((1,), (1,)), ((), ())),
                               preferred_element_type=jnp.float32)


def _ln(x, s, b):
    m = jnp.mean(x, axis=1, keepdims=True)
    v = jnp.mean((x - m) * (x - m), axis=1, keepdims=True)
    return (x - m) * jax.lax.rsqrt(v + 1e-5) * s + b


def _body(x_ref, am_ref, Ww_ref, bw_ref, Wa1_ref, ba1_ref, Wa2_ref, ba2_ref,
          ln1s_ref, ln1b_ref, Wqkv_ref, bqkv_ref, Wo_ref, bo_ref,
          ln2s_ref, ln2b_ref, Wm1_ref, bm1_ref, Wm2_ref, bm2_ref,
          out_ref, avg_ref):
    b_idx = pl.program_id(0)
    x = x_ref[0]
    ami = am_ref[0, 0]

    w = jnp.sum(x * Ww_ref[...], axis=1, keepdims=True) + bw_ref[0, 0]
    a1 = _dot(x, Wa1_ref[...]) + ba1_ref[...]
    a1 = a1 / (1.0 + jnp.exp(-a1))
    lcol = _dot(a1, Wa2_ref[...]) + ba2_ref[...]
    mask_col = lcol[:, 1:2] > lcol[:, 0:1]
    mask_f = mask_col.astype(jnp.float32)
    cnt = jnp.sum(mask_f, axis=0, keepdims=True)
    nmask = jnp.float32(S) - cnt

    a = _ln(x, ln1s_ref[...], ln1b_ref[...]).astype(BF)
    qkv = _dot(a, Wqkv_ref[...]) + bqkv_ref[...]

    scale = jnp.float32(1.0 / (DH ** 0.5))
    q_all = (qkv[:, 0:D] * scale).astype(BF)
    k_all = (qkv[:, D:2 * D] * mask_f).astype(BF)
    v_all = (qkv[:, 2 * D:3 * D] * mask_f).astype(BF)
    o_heads = []
    for h in range(H):
        q = q_all[:, h * DH:(h + 1) * DH]
        k = k_all[:, h * DH:(h + 1) * DH]
        v = v_all[:, h * DH:(h + 1) * DH]
        p = jnp.exp(_dot(q, k))
        r = 1.0 / (jnp.sum(p, axis=1, keepdims=True) - nmask)
        o_heads.append(jax.lax.dot_general(
            p.astype(BF), v, (((1,), (0,)), ((), ())),
            preferred_element_type=jnp.float32) * r)
    o = jnp.concatenate(o_heads, axis=1).astype(BF)

    h1 = x + _dot(o, Wo_ref[...]) + bo_ref[...]
    m = _ln(h1, ln2s_ref[...], ln2b_ref[...]).astype(BF)
    g = _dot(m, Wm1_ref[...]) + bm1_ref[...]
    g = 0.5 * g * (1.0 + jnp.tanh(0.7978845608028654 * (g + 0.044715 * g * g * g)))
    h2 = h1 + _dot(g.astype(BF), Wm2_ref[...]) + bm2_ref[...]

    out_ref[0] = jnp.where(mask_col, h2 * w, x)

    @pl.when(b_idx == 0)
    def _():
        avg_ref[...] = jnp.zeros((1, 1), jnp.float32)
    avg_ref[...] += cnt * jnp.float32(1.0 / B)


def kernel(x, attention_mask, Ww, bw, Wk1, bk1, Wk2, bk2, Wa1, ba1, Wa2, ba2,
           ln1_s, ln1_b, Wqkv, bqkv, Wo, bo, ln2_s, ln2_b, Wm1, bm1, Wm2, bm2):
    del Wk1, bk1, Wk2, bk2

    full = lambda shape: pl.BlockSpec(shape, lambda b: (0,) * len(shape))
    in_specs = [
        pl.BlockSpec((1, S, D), lambda b: (b, 0, 0)),
        pl.BlockSpec((1, 1, 1, S), lambda b: (b, 0, 0, 0)),
        full((1, D)),
        full((1, 1)),
        full((D // 2, D)),
        full((1, D // 2)),
        full((2, D // 2)),
        full((1, 2)),
        full((1, D)),
        full((1, D)),
        full((3 * D, D)),
        full((1, 3 * D)),
        full((D, D)),
        full((1, D)),
        full((1, D)),
        full((1, D)),
        full((DFF, D)),
        full((1, DFF)),
        full((D, DFF)),
        full((1, D)),
    ]
    out_specs = [
        pl.BlockSpec((1, S, D), lambda b: (b, 0, 0)),
        pl.BlockSpec((1, 1), lambda b: (0, 0)),
    ]
    out, avg = pl.pallas_call(
        _body,
        grid=(B,),
        in_specs=in_specs,
        out_specs=out_specs,
        out_shape=[
            jax.ShapeDtypeStruct((B, S, D), jnp.float32),
            jax.ShapeDtypeStruct((1, 1), jnp.float32),
        ],
    )(x, attention_mask,
      Ww, bw.reshape(1, 1), Wa1, ba1.reshape(1, -1), Wa2, ba2.reshape(1, -1),
      ln1_s.reshape(1, -1), ln1_b.reshape(1, -1),
      Wqkv.astype(BF), bqkv.reshape(1, -1),
      Wo.astype(BF), bo.reshape(1, -1),
      ln2_s.reshape(1, -1), ln2_b.reshape(1, -1),
      Wm1.astype(BF), bm1.reshape(1, -1),
      Wm2.astype(BF), bm2.reshape(1, -1))
    return (out, avg.reshape(()))
